# baseline (speedup 1.0000x reference)
import functools
import math

import jax
import jax.numpy as jnp
from jax import lax
from jax.experimental import pallas as pl
from jax.experimental.pallas import tpu as pltpu
from jax.experimental.pallas import tpu_sc as plsc

_TINY = 1e-13
_MASK_VAL = -10000.0
_IDX_CHUNK = 96


def _sc_gather(table, bias, ids, n_ids, d):
    info = plsc.get_sparse_core_info()
    nc, ns = info.num_cores, info.num_subcores
    nw = nc * ns
    per_w = n_ids // nw
    chunks = per_w // _IDX_CHUNK
    mesh = plsc.VectorSubcoreMesh(core_axis_name="c", subcore_axis_name="s")

    @functools.partial(
        pl.kernel,
        mesh=mesh,
        out_type=[
            jax.ShapeDtypeStruct((n_ids, d), jnp.float32),
            jax.ShapeDtypeStruct((n_ids,), jnp.float32),
        ],
        scratch_types=[
            pltpu.VMEM((per_w,), jnp.int32),
            pltpu.VMEM((per_w, d), jnp.float32),
            pltpu.VMEM((per_w,), jnp.float32),
            pltpu.SemaphoreType.DMA,
            pltpu.SemaphoreType.DMA,
        ],
        compiler_params=pltpu.CompilerParams(use_tc_tiling_on_sc=False,
                                             needs_layout_passes=False),
    )
    def gather(table_hbm, bias_hbm, idx_hbm, w_out, b_out, idx_v, rows_v,
               bflat_v, sem_w, sem_b):
        wid = lax.axis_index("s") * nc + lax.axis_index("c")
        base = wid * per_w
        pltpu.sync_copy(idx_hbm.at[pl.ds(base, per_w)], idx_v)
        copies = []
        for c in range(chunks):
            sl = pl.ds(c * _IDX_CHUNK, _IDX_CHUNK)
            copies.append(pltpu.async_copy(
                table_hbm.at[idx_v.at[sl]], rows_v.at[sl], sem_w))
            copies.append(pltpu.async_copy(
                bias_hbm.at[idx_v.at[sl]], bflat_v.at[sl], sem_b))
        for cp in copies:
            cp.wait()
        pltpu.sync_copy(rows_v, w_out.at[pl.ds(base, per_w)])
        pltpu.sync_copy(bflat_v, b_out.at[pl.ds(base, per_w)])

    return gather(table, bias, ids)


def _tc_body(nt_ref, emb_ref, tw_ref, sw_ref, tb_ref, sb_ref, tgt_ref,
             sid_ref, out_ref, *, log_nw_p1):
    i = pl.program_id(0)
    nt = nt_ref[0, 0]

    emb = emb_ref[...]
    tw = tw_ref[...]
    sw = sw_ref[...]
    tb = tb_ref[...]
    sb = sb_ref[...]
    tgt = tgt_ref[...]
    sid = sid_ref[...]

    t = tgt.astype(jnp.float32)
    tp = jnp.log((t + 2.0) / (t + 1.0)) * (1.0 / log_nw_p1)
    tec = 1.0 - jnp.exp(nt * jnp.log(1.0 - tp))
    true_logits = (jnp.sum(tw * emb, axis=1, keepdims=True) + tb
                   - jnp.log(tec + _TINY))

    s = sid.astype(jnp.float32)
    sp = jnp.log((s + 2.0) / (s + 1.0)) * (1.0 / log_nw_p1)
    sec = 1.0 - jnp.exp(nt * jnp.log(1.0 - sp))
    col_adj = sb - jnp.log(sec + _TINY)

    logits = lax.dot_general(emb, sw, (((1,), (1,)), ((), ())),
                             preferred_element_type=jnp.float32)
    logits = logits + col_adj
    logits = jnp.where(tgt == sid, _MASK_VAL, logits)

    m = jnp.maximum(jnp.max(logits, axis=1, keepdims=True), true_logits)
    se = (jnp.sum(jnp.exp(logits - m), axis=1, keepdims=True)
          + jnp.exp(true_logits - m))
    lse = m + jnp.log(se)
    part = jnp.sum(lse - true_logits, axis=(0, 1), keepdims=True)

    @pl.when(i == 0)
    def _():
        out_ref[...] = jnp.zeros_like(part)

    out_ref[...] += part


def kernel(embeddings, softmax_w, softmax_b, targets, sampled_ids, num_tries):
    b, d = embeddings.shape
    v = softmax_w.shape[0]
    s = sampled_ids.shape[0]
    n_ids = b + s
    log_nw_p1 = math.log(v + 1)

    all_ids = jnp.concatenate([targets, sampled_ids]).astype(jnp.int32)
    all_w, all_b = _sc_gather(softmax_w, softmax_b, all_ids, n_ids, d)

    tb = all_b[:b].reshape(b, 1)
    sb = all_b[b:].reshape(1, s)
    tgt2 = targets.reshape(b, 1)
    sid2 = sampled_ids.reshape(1, s)
    nt = jnp.reshape(num_tries, (1, 1)).astype(jnp.float32)

    tile = 256
    grid = (b // tile,)
    out = pl.pallas_call(
        functools.partial(_tc_body, log_nw_p1=log_nw_p1),
        grid=grid,
        in_specs=[
            pl.BlockSpec(memory_space=pltpu.SMEM),
            pl.BlockSpec((tile, d), lambda i: (i, 0)),
            pl.BlockSpec((tile, d), lambda i: (i, 0)),
            pl.BlockSpec((s, d), lambda i: (b // s, 0)),
            pl.BlockSpec((tile, 1), lambda i: (i, 0)),
            pl.BlockSpec((1, s), lambda i: (0, 0)),
            pl.BlockSpec((tile, 1), lambda i: (i, 0)),
            pl.BlockSpec((1, s), lambda i: (0, 0)),
        ],
        out_specs=pl.BlockSpec((1, 1), lambda i: (0, 0)),
        out_shape=jax.ShapeDtypeStruct((1, 1), jnp.float32),
        compiler_params=pltpu.CompilerParams(
            dimension_semantics=("arbitrary",)),
    )(nt, embeddings, all_w, all_w, tb, sb, tgt2, sid2)
    return out[0, 0]


# speedup vs baseline: 1.0017x; 1.0017x over previous
import functools
import math

import jax
import jax.numpy as jnp
from jax import lax
from jax.experimental import pallas as pl
from jax.experimental.pallas import tpu as pltpu
from jax.experimental.pallas import tpu_sc as plsc

_TINY = 1e-13
_IDX_CHUNK = 96


def _sc_gather(table, bias, ids, n_ids, d, batch, n_sampled):
    v = table.shape[0]
    info = plsc.get_sparse_core_info()
    nc, ns = info.num_cores, info.num_subcores
    nw = nc * ns
    per_w = n_ids // nw
    chunks = per_w // _IDX_CHUNK
    tgt_per_w = batch // nw
    smp_per_s = n_sampled // ns
    nwords = ((v // 32 + ns * 8 - 1) // (ns * 8)) * (ns * 8)
    zslice = nwords // ns
    mesh = plsc.VectorSubcoreMesh(core_axis_name="c", subcore_axis_name="s")

    @functools.partial(
        pl.kernel,
        mesh=mesh,
        out_type=[
            jax.ShapeDtypeStruct((n_ids, d), jnp.float32),
            jax.ShapeDtypeStruct((n_ids,), jnp.float32),
            jax.ShapeDtypeStruct((batch,), jnp.float32),
        ],
        scratch_types=[
            pltpu.VMEM((per_w,), jnp.int32),
            pltpu.VMEM((per_w, d), jnp.float32),
            pltpu.VMEM((per_w,), jnp.float32),
            pltpu.VMEM((tgt_per_w,), jnp.int32),
            pltpu.VMEM((smp_per_s,), jnp.int32),
            pltpu.VMEM((smp_per_s,), jnp.int32),
            pltpu.VMEM((smp_per_s,), jnp.int32),
            pltpu.VMEM((tgt_per_w,), jnp.int32),
            pltpu.VMEM((tgt_per_w,), jnp.int32),
            pltpu.VMEM((tgt_per_w,), jnp.float32),
            pltpu.VMEM((zslice,), jnp.int32),
            pltpu.VMEM_SHARED((nwords,), jnp.int32),
            pltpu.SemaphoreType.DMA,
            pltpu.SemaphoreType.DMA,
        ],
        compiler_params=pltpu.CompilerParams(use_tc_tiling_on_sc=False,
                                             needs_layout_passes=False),
    )
    def gather(table_hbm, bias_hbm, idx_hbm, w_out, b_out, f_out,
               idx_v, rows_v, bflat_v, tgt_v, smp_v, sw_v, sv_v, tw_v,
               words_v, fl_v, z_v, bitmap, sem_w, sem_b):
        cid = lax.axis_index("c")
        sid = lax.axis_index("s")
        wid = sid * nc + cid
        base = wid * per_w

        pltpu.sync_copy(idx_hbm.at[pl.ds(base, per_w)], idx_v)
        copies = []
        for c in range(chunks):
            sl = pl.ds(c * _IDX_CHUNK, _IDX_CHUNK)
            copies.append(pltpu.async_copy(
                table_hbm.at[idx_v.at[sl]], rows_v.at[sl], sem_w))
            copies.append(pltpu.async_copy(
                bias_hbm.at[idx_v.at[sl]], bflat_v.at[sl], sem_b))

        for j in range(zslice // 16):
            z_v[pl.ds(j * 16, 16)] = jnp.zeros((16,), jnp.int32)
        pltpu.sync_copy(z_v, bitmap.at[pl.ds(sid * zslice, zslice)])
        plsc.subcore_barrier()
        pltpu.sync_copy(idx_hbm.at[pl.ds(batch + sid * smp_per_s,
                                         smp_per_s)], smp_v)
        one = jnp.ones((16,), jnp.int32)
        for j in range(smp_per_s // 16):
            sl = pl.ds(j * 16, 16)
            sids = smp_v[sl]
            sw_v[sl] = lax.shift_right_logical(sids, 5)
            sv_v[sl] = lax.shift_left(one, jnp.bitwise_and(sids, 31))
        pltpu.sync_copy(sv_v, bitmap.at[sw_v], add=True)
        plsc.subcore_barrier()

        pltpu.sync_copy(idx_hbm.at[pl.ds(wid * tgt_per_w, tgt_per_w)],
                        tgt_v)
        for j in range(tgt_per_w // 16):
            sl = pl.ds(j * 16, 16)
            tw_v[sl] = lax.shift_right_logical(tgt_v[sl], 5)
        pltpu.sync_copy(bitmap.at[tw_v], words_v)
        for j in range(tgt_per_w // 16):
            sl = pl.ds(j * 16, 16)
            bit = jnp.bitwise_and(tgt_v[sl], 31)
            f = jnp.bitwise_and(
                lax.shift_right_logical(words_v[sl], bit), 1)
            fl_v[sl] = f.astype(jnp.float32)
        pltpu.sync_copy(fl_v, f_out.at[pl.ds(wid * tgt_per_w, tgt_per_w)])

        for cp in copies:
            cp.wait()
        pltpu.sync_copy(rows_v, w_out.at[pl.ds(base, per_w)])
        pltpu.sync_copy(bflat_v, b_out.at[pl.ds(base, per_w)])

    return gather(table, bias, ids)


def _tc_body(nt_ref, emb_ref, tw_ref, swb_ref, tb_ref, sb_ref, tgt_ref,
             sid_ref, fl_ref, out_ref, *, log_nw_p1):
    i = pl.program_id(0)
    nt = nt_ref[0, 0]

    emb = emb_ref[...]
    tw = tw_ref[...]
    swb = swb_ref[...]
    tb = tb_ref[...]
    sb = sb_ref[...]
    tgt = tgt_ref[...]
    sid = sid_ref[...]
    fl = fl_ref[...]

    t = tgt.astype(jnp.float32)
    tp = jnp.log((t + 2.0) / (t + 1.0)) * (1.0 / log_nw_p1)
    tec = 1.0 - jnp.exp(nt * jnp.log(1.0 - tp))
    true_logits = (jnp.sum(tw * emb, axis=1, keepdims=True) + tb
                   - jnp.log(tec + _TINY))

    s = sid.astype(jnp.float32)
    sp = jnp.log((s + 2.0) / (s + 1.0)) * (1.0 / log_nw_p1)
    sec = 1.0 - jnp.exp(nt * jnp.log(1.0 - sp))
    col_adj = sb - jnp.log(sec + _TINY)
    mcol = jnp.max(col_adj)
    ecol = jnp.exp(col_adj)

    dots = lax.dot_general(emb.astype(jnp.bfloat16), swb,
                           (((1,), (1,)), ((), ())),
                           preferred_element_type=jnp.float32)
    mdot = jnp.max(dots, axis=1, keepdims=True)
    m = jnp.maximum(mdot + mcol, true_logits)
    q = jnp.exp(dots - m) * ecol
    se = (jnp.sum(q, axis=1, keepdims=True)
          + (1.0 - fl) * jnp.exp(true_logits - m))
    lse = m + jnp.log(se)
    part = jnp.sum(lse - true_logits, axis=(0, 1), keepdims=True)

    @pl.when(i == 0)
    def _():
        out_ref[...] = jnp.zeros_like(part)

    out_ref[...] += part


def kernel(embeddings, softmax_w, softmax_b, targets, sampled_ids, num_tries):
    b, d = embeddings.shape
    v = softmax_w.shape[0]
    s = sampled_ids.shape[0]
    n_ids = b + s
    log_nw_p1 = math.log(v + 1)

    all_ids = jnp.concatenate([targets, sampled_ids]).astype(jnp.int32)
    all_w, all_b, flags = _sc_gather(softmax_w, softmax_b, all_ids,
                                     n_ids, d, b, s)

    swb = all_w[b:].astype(jnp.bfloat16)
    tb = all_b[:b].reshape(b, 1)
    sb = all_b[b:].reshape(1, s)
    tgt2 = targets.reshape(b, 1)
    sid2 = sampled_ids.reshape(1, s)
    fl2 = flags.reshape(b, 1)
    nt = jnp.reshape(num_tries, (1, 1)).astype(jnp.float32)

    tile = 256
    grid = (b // tile,)
    out = pl.pallas_call(
        functools.partial(_tc_body, log_nw_p1=log_nw_p1),
        grid=grid,
        in_specs=[
            pl.BlockSpec(memory_space=pltpu.SMEM),
            pl.BlockSpec((tile, d), lambda i: (i, 0)),
            pl.BlockSpec((tile, d), lambda i: (i, 0)),
            pl.BlockSpec((s, d), lambda i: (0, 0)),
            pl.BlockSpec((tile, 1), lambda i: (i, 0)),
            pl.BlockSpec((1, s), lambda i: (0, 0)),
            pl.BlockSpec((tile, 1), lambda i: (i, 0)),
            pl.BlockSpec((1, s), lambda i: (0, 0)),
            pl.BlockSpec((tile, 1), lambda i: (i, 0)),
        ],
        out_specs=pl.BlockSpec((1, 1), lambda i: (0, 0)),
        out_shape=jax.ShapeDtypeStruct((1, 1), jnp.float32),
        compiler_params=pltpu.CompilerParams(
            dimension_semantics=("arbitrary",)),
    )(nt, embeddings, all_w, swb, tb, sb, tgt2, sid2, fl2)
    return out[0, 0]


# speedup vs baseline: 1.0065x; 1.0048x over previous
import functools
import math

import jax
import jax.numpy as jnp
from jax import lax
from jax.experimental import pallas as pl
from jax.experimental.pallas import tpu as pltpu
from jax.experimental.pallas import tpu_sc as plsc

_TINY = 1e-13
_IDX_CHUNK = 96


def _sc_gather(wpair, bias, ids, n_ids, d, batch, n_sampled):
    v2 = wpair.shape[0]
    info = plsc.get_sparse_core_info()
    nc, ns = info.num_cores, info.num_subcores
    nw = nc * ns
    per_w = n_ids // nw
    chunks = per_w // _IDX_CHUNK
    tgt_per_w = batch // nw
    smp_per_s = n_sampled // ns
    nwords = ((2 * v2 // 32 + ns * 8 - 1) // (ns * 8)) * (ns * 8)
    zslice = nwords // ns
    mesh = plsc.VectorSubcoreMesh(core_axis_name="c", subcore_axis_name="s")

    @functools.partial(
        pl.kernel,
        mesh=mesh,
        out_type=[
            jax.ShapeDtypeStruct((n_ids, 2 * d), jnp.float32),
            jax.ShapeDtypeStruct((n_ids,), jnp.float32),
            jax.ShapeDtypeStruct((batch,), jnp.float32),
        ],
        scratch_types=[
            pltpu.VMEM((per_w,), jnp.int32),
            pltpu.VMEM((per_w,), jnp.int32),
            pltpu.VMEM((per_w, 2 * d), jnp.float32),
            pltpu.VMEM((per_w,), jnp.float32),
            pltpu.VMEM((tgt_per_w,), jnp.int32),
            pltpu.VMEM((smp_per_s,), jnp.int32),
            pltpu.VMEM((smp_per_s,), jnp.int32),
            pltpu.VMEM((smp_per_s,), jnp.int32),
            pltpu.VMEM((tgt_per_w,), jnp.int32),
            pltpu.VMEM((tgt_per_w,), jnp.int32),
            pltpu.VMEM((tgt_per_w,), jnp.float32),
            pltpu.VMEM((zslice,), jnp.int32),
            pltpu.VMEM_SHARED((nwords,), jnp.int32),
            pltpu.SemaphoreType.DMA,
            pltpu.SemaphoreType.DMA,
        ],
        compiler_params=pltpu.CompilerParams(use_tc_tiling_on_sc=False,
                                             needs_layout_passes=False),
    )
    def gather(table_hbm, bias_hbm, idx_hbm, w_out, b_out, f_out,
               idx_v, widx_v, rows_v, bflat_v, tgt_v, smp_v, sw_v, sv_v,
               tw_v, words_v, fl_v, z_v, bitmap, sem_w, sem_b):
        cid = lax.axis_index("c")
        sid = lax.axis_index("s")
        wid = sid * nc + cid
        base = wid * per_w

        pltpu.sync_copy(idx_hbm.at[pl.ds(base, per_w)], idx_v)
        for j in range(per_w // 16):
            sl = pl.ds(j * 16, 16)
            widx_v[sl] = lax.shift_right_logical(idx_v[sl], 1)
        copies = []
        for c in range(chunks):
            sl = pl.ds(c * _IDX_CHUNK, _IDX_CHUNK)
            copies.append(pltpu.async_copy(
                table_hbm.at[widx_v.at[sl]], rows_v.at[sl], sem_w))
            copies.append(pltpu.async_copy(
                bias_hbm.at[idx_v.at[sl]], bflat_v.at[sl], sem_b))

        for j in range(zslice // 16):
            z_v[pl.ds(j * 16, 16)] = jnp.zeros((16,), jnp.int32)
        pltpu.sync_copy(z_v, bitmap.at[pl.ds(sid * zslice, zslice)])
        plsc.subcore_barrier()
        pltpu.sync_copy(idx_hbm.at[pl.ds(batch + sid * smp_per_s,
                                         smp_per_s)], smp_v)
        one = jnp.ones((16,), jnp.int32)
        for j in range(smp_per_s // 16):
            sl = pl.ds(j * 16, 16)
            sids = smp_v[sl]
            sw_v[sl] = lax.shift_right_logical(sids, 5)
            sv_v[sl] = lax.shift_left(one, jnp.bitwise_and(sids, 31))
        pltpu.sync_copy(sv_v, bitmap.at[sw_v], add=True)
        plsc.subcore_barrier()

        pltpu.sync_copy(idx_hbm.at[pl.ds(wid * tgt_per_w, tgt_per_w)],
                        tgt_v)
        for j in range(tgt_per_w // 16):
            sl = pl.ds(j * 16, 16)
            tw_v[sl] = lax.shift_right_logical(tgt_v[sl], 5)
        pltpu.sync_copy(bitmap.at[tw_v], words_v)
        for j in range(tgt_per_w // 16):
            sl = pl.ds(j * 16, 16)
            bit = jnp.bitwise_and(tgt_v[sl], 31)
            f = jnp.bitwise_and(
                lax.shift_right_logical(words_v[sl], bit), 1)
            fl_v[sl] = f.astype(jnp.float32)
        pltpu.sync_copy(fl_v, f_out.at[pl.ds(wid * tgt_per_w, tgt_per_w)])

        for cp in copies:
            cp.wait()
        pltpu.sync_copy(rows_v, w_out.at[pl.ds(base, per_w)])
        pltpu.sync_copy(bflat_v, b_out.at[pl.ds(base, per_w)])

    return gather(wpair, bias, ids)


def _tc_body(nt_ref, emb_ref, twp_ref, swp_ref, tb_ref, sb_ref, tgt_ref,
             sid_ref, sidc_ref, fl_ref, out_ref, sws_ref, *, log_nw_p1, d):
    i = pl.program_id(0)
    nt = nt_ref[0, 0]

    emb = emb_ref[...]
    twp = twp_ref[...]
    tb = tb_ref[...]
    sb = sb_ref[...]
    tgt = tgt_ref[...]
    sid = sid_ref[...]
    fl = fl_ref[...]

    @pl.when(i == 0)
    def _():
        swp = swp_ref[...]
        sidc = sidc_ref[...]
        odd = jnp.bitwise_and(sidc, 1) == 1
        sel = jnp.where(odd, swp[:, d:2 * d], swp[:, 0:d])
        sws_ref[...] = sel.astype(jnp.bfloat16)

    todd = jnp.bitwise_and(tgt, 1) == 1
    tw = jnp.where(todd, twp[:, d:2 * d], twp[:, 0:d])

    t = tgt.astype(jnp.float32)
    tp = jnp.log((t + 2.0) / (t + 1.0)) * (1.0 / log_nw_p1)
    tec = 1.0 - jnp.exp(nt * jnp.log(1.0 - tp))
    true_logits = (jnp.sum(tw * emb, axis=1, keepdims=True) + tb
                   - jnp.log(tec + _TINY))

    s = sid.astype(jnp.float32)
    sp = jnp.log((s + 2.0) / (s + 1.0)) * (1.0 / log_nw_p1)
    sec = 1.0 - jnp.exp(nt * jnp.log(1.0 - sp))
    col_adj = sb - jnp.log(sec + _TINY)
    mcol = jnp.max(col_adj)
    ecol = jnp.exp(col_adj)

    dots = lax.dot_general(emb.astype(jnp.bfloat16), sws_ref[...],
                           (((1,), (1,)), ((), ())),
                           preferred_element_type=jnp.float32)
    mdot = jnp.max(dots, axis=1, keepdims=True)
    m = jnp.maximum(mdot + mcol, true_logits)
    q = jnp.exp(dots - m) * ecol
    se = (jnp.sum(q, axis=1, keepdims=True)
          + (1.0 - fl) * jnp.exp(true_logits - m))
    lse = m + jnp.log(se)
    part = jnp.sum(lse - true_logits, axis=(0, 1), keepdims=True)

    @pl.when(i == 0)
    def _():
        out_ref[...] = jnp.zeros_like(part)

    out_ref[...] += part


def kernel(embeddings, softmax_w, softmax_b, targets, sampled_ids, num_tries):
    b, d = embeddings.shape
    v = softmax_w.shape[0]
    s = sampled_ids.shape[0]
    n_ids = b + s
    log_nw_p1 = math.log(v + 1)

    wpair = softmax_w.reshape(v // 2, 2 * d)
    all_ids = jnp.concatenate([targets, sampled_ids]).astype(jnp.int32)
    all_wp, all_b, flags = _sc_gather(wpair, softmax_b, all_ids,
                                      n_ids, d, b, s)

    tb = all_b[:b].reshape(b, 1)
    sb = all_b[b:].reshape(1, s)
    tgt2 = targets.reshape(b, 1)
    sid2 = sampled_ids.reshape(1, s)
    sidc = sampled_ids.reshape(s, 1)
    fl2 = flags.reshape(b, 1)
    nt = jnp.reshape(num_tries, (1, 1)).astype(jnp.float32)

    tile = 256
    grid = (b // tile,)
    out = pl.pallas_call(
        functools.partial(_tc_body, log_nw_p1=log_nw_p1, d=d),
        grid=grid,
        in_specs=[
            pl.BlockSpec(memory_space=pltpu.SMEM),
            pl.BlockSpec((tile, d), lambda i: (i, 0)),
            pl.BlockSpec((tile, 2 * d), lambda i: (i, 0)),
            pl.BlockSpec((s, 2 * d), lambda i: (b // s, 0)),
            pl.BlockSpec((tile, 1), lambda i: (i, 0)),
            pl.BlockSpec((1, s), lambda i: (0, 0)),
            pl.BlockSpec((tile, 1), lambda i: (i, 0)),
            pl.BlockSpec((1, s), lambda i: (0, 0)),
            pl.BlockSpec((s, 1), lambda i: (0, 0)),
            pl.BlockSpec((tile, 1), lambda i: (i, 0)),
        ],
        out_specs=pl.BlockSpec((1, 1), lambda i: (0, 0)),
        out_shape=jax.ShapeDtypeStruct((1, 1), jnp.float32),
        scratch_shapes=[pltpu.VMEM((s, d), jnp.bfloat16)],
        compiler_params=pltpu.CompilerParams(
            dimension_semantics=("arbitrary",)),
    )(nt, embeddings, all_wp, all_wp, tb, sb, tgt2, sid2, sidc, fl2)
    return out[0, 0]


# speedup vs baseline: 1.0079x; 1.0015x over previous
import functools
import math

import jax
import jax.numpy as jnp
from jax import lax
from jax.experimental import pallas as pl
from jax.experimental.pallas import tpu as pltpu
from jax.experimental.pallas import tpu_sc as plsc

_TINY = 1e-13
_IDX_CHUNK = 96


def _sc_gather_rows(wpair, ids, n_ids, d):
    info = plsc.get_sparse_core_info()
    nc, ns = info.num_cores, info.num_subcores
    nw = nc * ns
    per_w = n_ids // nw
    chunks = per_w // _IDX_CHUNK
    mesh = plsc.VectorSubcoreMesh(core_axis_name="c", subcore_axis_name="s")

    @functools.partial(
        pl.kernel,
        mesh=mesh,
        out_type=jax.ShapeDtypeStruct((n_ids, 2 * d), jnp.float32),
        scratch_types=[
            pltpu.VMEM((per_w,), jnp.int32),
            pltpu.VMEM((per_w,), jnp.int32),
            pltpu.VMEM((per_w, 2 * d), jnp.float32),
            pltpu.SemaphoreType.DMA,
        ],
        compiler_params=pltpu.CompilerParams(use_tc_tiling_on_sc=True,
                                             needs_layout_passes=False),
    )
    def gather(table_hbm, idx_hbm, w_out, idx_v, widx_v, rows_v, sem_w):
        cid = lax.axis_index("c")
        sid = lax.axis_index("s")
        wid = sid * nc + cid
        base = wid * per_w
        pltpu.sync_copy(idx_hbm.at[pl.ds(base, per_w)], idx_v)
        for j in range(per_w // 16):
            sl = pl.ds(j * 16, 16)
            widx_v[sl] = lax.shift_right_logical(idx_v[sl], 1)
        copies = []
        for c in range(chunks):
            sl = pl.ds(c * _IDX_CHUNK, _IDX_CHUNK)
            copies.append(pltpu.async_copy(
                table_hbm.at[widx_v.at[sl]], rows_v.at[sl], sem_w))
        for cp in copies:
            cp.wait()
        pltpu.sync_copy(rows_v, w_out.at[pl.ds(base, per_w)])

    return gather(wpair, ids)


def _sc_bias_flags(bias, ids, n_ids, batch, n_sampled):
    v = bias.shape[0]
    info = plsc.get_sparse_core_info()
    nc, ns = info.num_cores, info.num_subcores
    nw = nc * ns
    per_w = n_ids // nw
    chunks = per_w // _IDX_CHUNK
    tgt_per_w = batch // nw
    smp_per_s = n_sampled // ns
    nwords = ((v // 32 + ns * 8 - 1) // (ns * 8)) * (ns * 8)
    zslice = nwords // ns
    mesh = plsc.VectorSubcoreMesh(core_axis_name="c", subcore_axis_name="s")

    @functools.partial(
        pl.kernel,
        mesh=mesh,
        out_type=[
            jax.ShapeDtypeStruct((n_ids,), jnp.float32),
            jax.ShapeDtypeStruct((batch,), jnp.float32),
        ],
        scratch_types=[
            pltpu.VMEM((per_w,), jnp.int32),
            pltpu.VMEM((per_w,), jnp.float32),
            pltpu.VMEM((tgt_per_w,), jnp.int32),
            pltpu.VMEM((smp_per_s,), jnp.int32),
            pltpu.VMEM((smp_per_s,), jnp.int32),
            pltpu.VMEM((smp_per_s,), jnp.int32),
            pltpu.VMEM((tgt_per_w,), jnp.int32),
            pltpu.VMEM((tgt_per_w,), jnp.int32),
            pltpu.VMEM((tgt_per_w,), jnp.float32),
            pltpu.VMEM((zslice,), jnp.int32),
            pltpu.VMEM_SHARED((nwords,), jnp.int32),
            pltpu.SemaphoreType.DMA,
        ],
        compiler_params=pltpu.CompilerParams(use_tc_tiling_on_sc=False,
                                             needs_layout_passes=False),
    )
    def gather(bias_hbm, idx_hbm, b_out, f_out,
               idx_v, bflat_v, tgt_v, smp_v, sw_v, sv_v,
               tw_v, words_v, fl_v, z_v, bitmap, sem_b):
        cid = lax.axis_index("c")
        sid = lax.axis_index("s")
        wid = sid * nc + cid
        base = wid * per_w

        pltpu.sync_copy(idx_hbm.at[pl.ds(base, per_w)], idx_v)
        copies = []
        for c in range(chunks):
            sl = pl.ds(c * _IDX_CHUNK, _IDX_CHUNK)
            copies.append(pltpu.async_copy(
                bias_hbm.at[idx_v.at[sl]], bflat_v.at[sl], sem_b))

        for j in range(zslice // 16):
            z_v[pl.ds(j * 16, 16)] = jnp.zeros((16,), jnp.int32)
        pltpu.sync_copy(z_v, bitmap.at[pl.ds(sid * zslice, zslice)])
        plsc.subcore_barrier()
        pltpu.sync_copy(idx_hbm.at[pl.ds(batch + sid * smp_per_s,
                                         smp_per_s)], smp_v)
        one = jnp.ones((16,), jnp.int32)
        for j in range(smp_per_s // 16):
            sl = pl.ds(j * 16, 16)
            sids = smp_v[sl]
            sw_v[sl] = lax.shift_right_logical(sids, 5)
            sv_v[sl] = lax.shift_left(one, jnp.bitwise_and(sids, 31))
        pltpu.sync_copy(sv_v, bitmap.at[sw_v], add=True)
        plsc.subcore_barrier()

        pltpu.sync_copy(idx_hbm.at[pl.ds(wid * tgt_per_w, tgt_per_w)],
                        tgt_v)
        for j in range(tgt_per_w // 16):
            sl = pl.ds(j * 16, 16)
            tw_v[sl] = lax.shift_right_logical(tgt_v[sl], 5)
        pltpu.sync_copy(bitmap.at[tw_v], words_v)
        for j in range(tgt_per_w // 16):
            sl = pl.ds(j * 16, 16)
            bit = jnp.bitwise_and(tgt_v[sl], 31)
            f = jnp.bitwise_and(
                lax.shift_right_logical(words_v[sl], bit), 1)
            fl_v[sl] = f.astype(jnp.float32)
        pltpu.sync_copy(fl_v, f_out.at[pl.ds(wid * tgt_per_w, tgt_per_w)])

        for cp in copies:
            cp.wait()
        pltpu.sync_copy(bflat_v, b_out.at[pl.ds(base, per_w)])

    return gather(bias, ids)


def _tc_body(nt_ref, emb_ref, twp_ref, swp_ref, tb_ref, sb_ref, tgt_ref,
             sid_ref, sidc_ref, fl_ref, out_ref, sws_ref, *, log_nw_p1, d):
    i = pl.program_id(0)
    nt = nt_ref[0, 0]

    emb = emb_ref[...]
    twp = twp_ref[...]
    tb = tb_ref[...]
    sb = sb_ref[...]
    tgt = tgt_ref[...]
    sid = sid_ref[...]
    fl = fl_ref[...]

    @pl.when(i == 0)
    def _():
        swp = swp_ref[...]
        sidc = sidc_ref[...]
        odd = jnp.bitwise_and(sidc, 1) == 1
        sel = jnp.where(odd, swp[:, d:2 * d], swp[:, 0:d])
        sws_ref[...] = sel.astype(jnp.bfloat16)

    todd = jnp.bitwise_and(tgt, 1) == 1
    tw = jnp.where(todd, twp[:, d:2 * d], twp[:, 0:d])

    t = tgt.astype(jnp.float32)
    tp = jnp.log((t + 2.0) / (t + 1.0)) * (1.0 / log_nw_p1)
    tec = 1.0 - jnp.exp(nt * jnp.log(1.0 - tp))
    true_logits = (jnp.sum(tw * emb, axis=1, keepdims=True) + tb
                   - jnp.log(tec + _TINY))

    s = sid.astype(jnp.float32)
    sp = jnp.log((s + 2.0) / (s + 1.0)) * (1.0 / log_nw_p1)
    sec = 1.0 - jnp.exp(nt * jnp.log(1.0 - sp))
    col_adj = sb - jnp.log(sec + _TINY)
    mcol = jnp.max(col_adj)
    ecol = jnp.exp(col_adj)

    dots = lax.dot_general(emb.astype(jnp.bfloat16), sws_ref[...],
                           (((1,), (1,)), ((), ())),
                           preferred_element_type=jnp.float32)
    mdot = jnp.max(dots, axis=1, keepdims=True)
    m = jnp.maximum(mdot + mcol, true_logits)
    q = jnp.exp(dots - m) * ecol
    se = (jnp.sum(q, axis=1, keepdims=True)
          + (1.0 - fl) * jnp.exp(true_logits - m))
    lse = m + jnp.log(se)
    part = jnp.sum(lse - true_logits, axis=(0, 1), keepdims=True)

    @pl.when(i == 0)
    def _():
        out_ref[...] = jnp.zeros_like(part)

    out_ref[...] += part


def kernel(embeddings, softmax_w, softmax_b, targets, sampled_ids, num_tries):
    b, d = embeddings.shape
    v = softmax_w.shape[0]
    s = sampled_ids.shape[0]
    n_ids = b + s
    log_nw_p1 = math.log(v + 1)

    wpair = softmax_w.reshape(v // 2, 2 * d)
    all_ids = jnp.concatenate([targets, sampled_ids]).astype(jnp.int32)
    all_wp = _sc_gather_rows(wpair, all_ids, n_ids, d)
    all_b, flags = _sc_bias_flags(softmax_b, all_ids, n_ids, b, s)

    tb = all_b[:b].reshape(b, 1)
    sb = all_b[b:].reshape(1, s)
    tgt2 = targets.reshape(b, 1)
    sid2 = sampled_ids.reshape(1, s)
    sidc = sampled_ids.reshape(s, 1)
    fl2 = flags.reshape(b, 1)
    nt = jnp.reshape(num_tries, (1, 1)).astype(jnp.float32)

    tile = 256
    grid = (b // tile,)
    out = pl.pallas_call(
        functools.partial(_tc_body, log_nw_p1=log_nw_p1, d=d),
        grid=grid,
        in_specs=[
            pl.BlockSpec(memory_space=pltpu.SMEM),
            pl.BlockSpec((tile, d), lambda i: (i, 0)),
            pl.BlockSpec((tile, 2 * d), lambda i: (i, 0)),
            pl.BlockSpec((s, 2 * d), lambda i: (b // s, 0)),
            pl.BlockSpec((tile, 1), lambda i: (i, 0)),
            pl.BlockSpec((1, s), lambda i: (0, 0)),
            pl.BlockSpec((tile, 1), lambda i: (i, 0)),
            pl.BlockSpec((1, s), lambda i: (0, 0)),
            pl.BlockSpec((s, 1), lambda i: (0, 0)),
            pl.BlockSpec((tile, 1), lambda i: (i, 0)),
        ],
        out_specs=pl.BlockSpec((1, 1), lambda i: (0, 0)),
        out_shape=jax.ShapeDtypeStruct((1, 1), jnp.float32),
        scratch_shapes=[pltpu.VMEM((s, d), jnp.bfloat16)],
        compiler_params=pltpu.CompilerParams(
            dimension_semantics=("arbitrary",)),
    )(nt, embeddings, all_wp, all_wp, tb, sb, tgt2, sid2, sidc, fl2)
    return out[0, 0]


# speedup vs baseline: 1.0961x; 1.0875x over previous
import functools
import math

import jax
import jax.numpy as jnp
from jax import lax
from jax.experimental import pallas as pl
from jax.experimental.pallas import tpu as pltpu
from jax.experimental.pallas import tpu_sc as plsc

_TINY = 1e-13
_IDX_CHUNK = 96


def _sc_gather_rows(wpair, ids, n_ids, d):
    info = plsc.get_sparse_core_info()
    nc, ns = info.num_cores, info.num_subcores
    nw = nc * ns
    per_w = n_ids // nw
    chunks = per_w // _IDX_CHUNK
    mesh = plsc.VectorSubcoreMesh(core_axis_name="c", subcore_axis_name="s")

    @functools.partial(
        pl.kernel,
        mesh=mesh,
        out_type=jax.ShapeDtypeStruct((n_ids, 2 * d), jnp.float32),
        scratch_types=[
            pltpu.VMEM((per_w,), jnp.int32),
            pltpu.VMEM((per_w,), jnp.int32),
            pltpu.VMEM((per_w, 2 * d), jnp.float32),
            pltpu.SemaphoreType.DMA,
        ],
        compiler_params=pltpu.CompilerParams(use_tc_tiling_on_sc=True,
                                             needs_layout_passes=False),
    )
    def gather(table_hbm, idx_hbm, w_out, idx_v, widx_v, rows_v, sem_w):
        del widx_v
        cid = lax.axis_index("c")
        sid = lax.axis_index("s")
        wid = sid * nc + cid
        base = wid * per_w
        pltpu.sync_copy(idx_hbm.at[pl.ds(base, per_w)], idx_v)
        copies = []
        for c in range(chunks):
            sl = pl.ds(c * _IDX_CHUNK, _IDX_CHUNK)
            copies.append(pltpu.async_copy(
                table_hbm.at[idx_v.at[sl]], rows_v.at[sl], sem_w))
        for cp in copies:
            cp.wait()
        pltpu.sync_copy(rows_v, w_out.at[pl.ds(base, per_w)])

    return gather(wpair, ids)


def _sc_bias_flags(bias, ids, n_ids, batch, n_sampled):
    v = bias.shape[0]
    info = plsc.get_sparse_core_info()
    nc, ns = info.num_cores, info.num_subcores
    nw = nc * ns
    per_w = n_ids // nw
    chunks = per_w // _IDX_CHUNK
    tgt_per_w = batch // nw
    smp_per_s = n_sampled // ns
    nwords = ((v // 32 + ns * 8 - 1) // (ns * 8)) * (ns * 8)
    zslice = nwords // ns
    mesh = plsc.VectorSubcoreMesh(core_axis_name="c", subcore_axis_name="s")

    @functools.partial(
        pl.kernel,
        mesh=mesh,
        out_type=[
            jax.ShapeDtypeStruct((n_ids,), jnp.float32),
            jax.ShapeDtypeStruct((batch,), jnp.float32),
        ],
        scratch_types=[
            pltpu.VMEM((per_w,), jnp.int32),
            pltpu.VMEM((per_w,), jnp.float32),
            pltpu.VMEM((tgt_per_w,), jnp.int32),
            pltpu.VMEM((smp_per_s,), jnp.int32),
            pltpu.VMEM((smp_per_s,), jnp.int32),
            pltpu.VMEM((smp_per_s,), jnp.int32),
            pltpu.VMEM((tgt_per_w,), jnp.int32),
            pltpu.VMEM((tgt_per_w,), jnp.int32),
            pltpu.VMEM((tgt_per_w,), jnp.float32),
            pltpu.VMEM((zslice,), jnp.int32),
            pltpu.VMEM_SHARED((nwords,), jnp.int32),
            pltpu.SemaphoreType.DMA,
        ],
        compiler_params=pltpu.CompilerParams(use_tc_tiling_on_sc=False,
                                             needs_layout_passes=False),
    )
    def gather(bias_hbm, idx_hbm, b_out, f_out,
               idx_v, bflat_v, tgt_v, smp_v, sw_v, sv_v,
               tw_v, words_v, fl_v, z_v, bitmap, sem_b):
        cid = lax.axis_index("c")
        sid = lax.axis_index("s")
        wid = sid * nc + cid
        base = wid * per_w

        pltpu.sync_copy(idx_hbm.at[pl.ds(base, per_w)], idx_v)
        copies = []
        for c in range(chunks):
            sl = pl.ds(c * _IDX_CHUNK, _IDX_CHUNK)
            copies.append(pltpu.async_copy(
                bias_hbm.at[idx_v.at[sl]], bflat_v.at[sl], sem_b))

        for j in range(zslice // 16):
            z_v[pl.ds(j * 16, 16)] = jnp.zeros((16,), jnp.int32)
        pltpu.sync_copy(z_v, bitmap.at[pl.ds(sid * zslice, zslice)])
        plsc.subcore_barrier()
        pltpu.sync_copy(idx_hbm.at[pl.ds(batch + sid * smp_per_s,
                                         smp_per_s)], smp_v)
        one = jnp.ones((16,), jnp.int32)
        for j in range(smp_per_s // 16):
            sl = pl.ds(j * 16, 16)
            sids = smp_v[sl]
            sw_v[sl] = lax.shift_right_logical(sids, 5)
            sv_v[sl] = lax.shift_left(one, jnp.bitwise_and(sids, 31))
        pltpu.sync_copy(sv_v, bitmap.at[sw_v], add=True)
        plsc.subcore_barrier()

        pltpu.sync_copy(idx_hbm.at[pl.ds(wid * tgt_per_w, tgt_per_w)],
                        tgt_v)
        for j in range(tgt_per_w // 16):
            sl = pl.ds(j * 16, 16)
            tw_v[sl] = lax.shift_right_logical(tgt_v[sl], 5)
        pltpu.sync_copy(bitmap.at[tw_v], words_v)
        for j in range(tgt_per_w // 16):
            sl = pl.ds(j * 16, 16)
            bit = jnp.bitwise_and(tgt_v[sl], 31)
            f = jnp.bitwise_and(
                lax.shift_right_logical(words_v[sl], bit), 1)
            fl_v[sl] = f.astype(jnp.float32)
        pltpu.sync_copy(fl_v, f_out.at[pl.ds(wid * tgt_per_w, tgt_per_w)])

        for cp in copies:
            cp.wait()
        pltpu.sync_copy(bflat_v, b_out.at[pl.ds(base, per_w)])

    return gather(bias, ids)


def _tc_body(nt_ref, emb_ref, twp_ref, swp_ref, tb_ref, sb_ref, tgt_ref,
             sid_ref, fl_ref, out_ref, sws_ref, *, log_nw_p1, d):
    i = pl.program_id(0)
    nt = nt_ref[0, 0]

    emb = emb_ref[...]
    twp = twp_ref[...]
    tb = tb_ref[...]
    sb = sb_ref[...]
    tgt = tgt_ref[...]
    sid = sid_ref[...]
    fl = fl_ref[...]

    @pl.when(i == 0)
    def _():
        swp = swp_ref[...]
        sws_ref[...] = swp[:, 0:d].astype(jnp.bfloat16)

    tw = twp[:, 0:d]

    t = tgt.astype(jnp.float32)
    tp = jnp.log((t + 2.0) / (t + 1.0)) * (1.0 / log_nw_p1)
    tec = 1.0 - jnp.exp(nt * jnp.log(1.0 - tp))
    true_logits = (jnp.sum(tw * emb, axis=1, keepdims=True) + tb
                   - jnp.log(tec + _TINY))

    s = sid.astype(jnp.float32)
    sp = jnp.log((s + 2.0) / (s + 1.0)) * (1.0 / log_nw_p1)
    sec = 1.0 - jnp.exp(nt * jnp.log(1.0 - sp))
    col_adj = sb - jnp.log(sec + _TINY)
    mcol = jnp.max(col_adj)
    ecol = jnp.exp(col_adj)

    dots = lax.dot_general(emb.astype(jnp.bfloat16), sws_ref[...],
                           (((1,), (1,)), ((), ())),
                           preferred_element_type=jnp.float32)
    mdot = jnp.max(dots, axis=1, keepdims=True)
    m = jnp.maximum(mdot + mcol, true_logits)
    q = jnp.exp(dots - m) * ecol
    se = (jnp.sum(q, axis=1, keepdims=True)
          + (1.0 - fl) * jnp.exp(true_logits - m))
    lse = m + jnp.log(se)
    part = jnp.sum(lse - true_logits, axis=(0, 1), keepdims=True)

    @pl.when(i == 0)
    def _():
        out_ref[...] = jnp.zeros_like(part)

    out_ref[...] += part


def kernel(embeddings, softmax_w, softmax_b, targets, sampled_ids, num_tries):
    b, d = embeddings.shape
    v = softmax_w.shape[0]
    s = sampled_ids.shape[0]
    n_ids = b + s
    log_nw_p1 = math.log(v + 1)

    wpad = jnp.pad(softmax_w, ((0, 0), (0, d)))
    all_ids = jnp.concatenate([targets, sampled_ids]).astype(jnp.int32)
    all_wp = _sc_gather_rows(wpad, all_ids, n_ids, d)
    all_b, flags = _sc_bias_flags(softmax_b, all_ids, n_ids, b, s)

    tb = all_b[:b].reshape(b, 1)
    sb = all_b[b:].reshape(1, s)
    tgt2 = targets.reshape(b, 1)
    sid2 = sampled_ids.reshape(1, s)
    fl2 = flags.reshape(b, 1)
    nt = jnp.reshape(num_tries, (1, 1)).astype(jnp.float32)

    tile = 256
    grid = (b // tile,)
    out = pl.pallas_call(
        functools.partial(_tc_body, log_nw_p1=log_nw_p1, d=d),
        grid=grid,
        in_specs=[
            pl.BlockSpec(memory_space=pltpu.SMEM),
            pl.BlockSpec((tile, d), lambda i: (i, 0)),
            pl.BlockSpec((tile, 2 * d), lambda i: (i, 0)),
            pl.BlockSpec((s, 2 * d), lambda i: (b // s, 0)),
            pl.BlockSpec((tile, 1), lambda i: (i, 0)),
            pl.BlockSpec((1, s), lambda i: (0, 0)),
            pl.BlockSpec((tile, 1), lambda i: (i, 0)),
            pl.BlockSpec((1, s), lambda i: (0, 0)),
            pl.BlockSpec((tile, 1), lambda i: (i, 0)),
        ],
        out_specs=pl.BlockSpec((1, 1), lambda i: (0, 0)),
        out_shape=jax.ShapeDtypeStruct((1, 1), jnp.float32),
        scratch_shapes=[pltpu.VMEM((s, d), jnp.bfloat16)],
        compiler_params=pltpu.CompilerParams(
            dimension_semantics=("arbitrary",)),
    )(nt, embeddings, all_wp, all_wp, tb, sb, tgt2, sid2, fl2)
    return out[0, 0]
